# baseline (speedup 1.0000x reference)
import jax
import jax.numpy as jnp
from jax import lax
from jax.experimental import pallas as pl
from jax.experimental.pallas import tpu as pltpu, tpu_sc as plsc

B = 4096
F = 26
V = 100000
D = 32
L = 50
W = D // 2

LSEQ = 64
CHUNK = 32
SP_SLOTS = 27
SP_IDX = CHUNK * SP_SLOTS
SP_IDX_PAD = 1024
SEQ_IDX = CHUNK * LSEQ
N_CHUNKS = B // CHUNK

VSPAN = 1024
NFULL = V // VSPAN
TAIL0 = NFULL * VSPAN
TAIL1 = TAIL0 + 5 * 128
N_ITEMS = F * NFULL
VP = 100096
N_WORKERS = 32
ITEMS_PER_W = -(-N_ITEMS // N_WORKERS)


def _transpose_block(tb, ob, width, iota2a, iota2b):
    def tr(u, col):
        for j in range(8):
            a = plsc.load_gather(tb, [iota2a, col])
            bvals = plsc.load_gather(tb, [iota2b, col])
            ob[u, pl.ds(j * W, W)] = plsc.bitcast(
                plsc.pack(a, bvals, format=plsc.PackFormat.INTERLEAVED),
                jnp.int32)
            col = col + 1
        return col
    lax.fori_loop(0, width // 8, tr, jnp.zeros((16,), jnp.int32))


def _kernel_a1(t2_hbm, cpad_hbm, tflat_hbm,
               tb0, tb1, ob0, ob1, tbe, obe, sem_t, sem_o):
    info = plsc.get_sparse_core_info()
    nc = info.num_cores
    wid = lax.axis_index("s") * nc + lax.axis_index("c")
    iota = lax.iota(jnp.int32, 16)
    iota2a = iota
    iota2b = iota + 16

    @pl.when(wid < F)
    def _edge():
        f = wid
        for kk in range(5):
            pltpu.sync_copy(
                t2_hbm.at[pl.ds(f * D, D), pl.ds(TAIL0 + kk * 128, 128)],
                tbe.at[:, pl.ds(0, 128)])
            _transpose_block(tbe, obe, 128, iota2a, iota2b)
            pltpu.sync_copy(
                obe.at[pl.ds(0, 16)],
                tflat_hbm.at[pl.ds(pl.multiple_of(
                    f * (VP // 8) + (TAIL0 + kk * 128) // 8, 8), 16)])
        pltpu.sync_copy(cpad_hbm.at[pl.ds(f * 32, 32), pl.ds(0, 128)], tbe)

        def crow(j, carry):
            jf = jnp.full((16,), j, jnp.int32)
            a = plsc.load_gather(tbe, [jf, iota2a])
            bvals = plsc.load_gather(tbe, [jf, iota2b])
            packed = plsc.bitcast(
                plsc.pack(a, bvals, format=plsc.PackFormat.INTERLEAVED),
                jnp.int32)
            obe[j // 8, pl.ds((j % 8) * W, W)] = packed
            return carry
        lax.fori_loop(0, 32, crow, 0)
        pltpu.sync_copy(obe.at[pl.ds(0, 4)],
                        tflat_hbm.at[pl.ds(pl.multiple_of(f * (VP // 8) + TAIL1 // 8, 8), 4)])

    def item_of(i):
        item = i * N_WORKERS + wid
        return jnp.minimum(item, N_ITEMS - 1), item < N_ITEMS

    def stage_item(i, tb):
        item, _ = item_of(i)
        f = item // NFULL
        s = item % NFULL
        for dt in range(4):
            pltpu.async_copy(
                t2_hbm.at[pl.ds(pl.multiple_of(f * D + dt * 8, 8), 8),
                          pl.ds(s * VSPAN, VSPAN)],
                tb.at[pl.ds(dt * 8, 8)], sem_t)

    def drain_tb(tb):
        for dt in range(4):
            pltpu.make_async_copy(
                t2_hbm.at[pl.ds(0, 8), pl.ds(0, VSPAN)],
                tb.at[pl.ds(dt * 8, 8)], sem_t).wait()

    def drain_ob(ob):
        pltpu.make_async_copy(
            tflat_hbm.at[pl.ds(0, VSPAN // 8)], ob, sem_o).wait()

    def do_item(i, tb, ob):
        item, valid = item_of(i)

        @pl.when(valid)
        def _():
            f = item // NFULL
            s = item % NFULL
            _transpose_block(tb, ob, VSPAN, iota2a, iota2b)
            pltpu.async_copy(
                ob, tflat_hbm.at[pl.ds(pl.multiple_of(f * (VP // 8) + s * (VSPAN // 8), 8),
                                   VSPAN // 8)],
                sem_o)

    stage_item(0, tb0)

    def pipe(k, carry):
        @pl.when(k % 2 == 0)
        def _ev():
            drain_tb(tb0)
            stage_item(k + 1, tb1)
            @pl.when(jnp.logical_and(k >= 2, (k - 2) * N_WORKERS + wid < N_ITEMS))
            def _d0():
                drain_ob(ob0)
            do_item(k, tb0, ob0)

        @pl.when(k % 2 == 1)
        def _od():
            drain_tb(tb1)
            stage_item(k + 1, tb0)
            @pl.when(jnp.logical_and(k >= 2, (k - 2) * N_WORKERS + wid < N_ITEMS))
            def _d1():
                drain_ob(ob1)
            do_item(k, tb1, ob1)

        return carry

    lax.fori_loop(0, ITEMS_PER_W, pipe, 0)
    if ITEMS_PER_W % 2 == 0:
        drain_tb(tb0)
    else:
        drain_tb(tb1)
    for kk in (ITEMS_PER_W - 2, ITEMS_PER_W - 1):
        @pl.when(kk * N_WORKERS + wid < N_ITEMS)
        def _dt(kk=kk):
            drain_ob(ob0 if kk % 2 == 0 else ob1)


def _kernel_a2(idx_seq_hbm, seq16_hbm, pooled_hbm,
               seq_dst, idx_seq_v, pbuf, sem_seq):
    info = plsc.get_sparse_core_info()
    nc = info.num_cores
    wid = lax.axis_index("s") * nc + lax.axis_index("c")
    n_workers = nc * info.num_subcores
    chunks_per_w = N_CHUNKS // n_workers

    def do_chunk(ci, carry):
        g = wid * chunks_per_w + ci
        pltpu.sync_copy(idx_seq_hbm.at[pl.ds(g * (SEQ_IDX // 128),
                                             SEQ_IDX // 128)], idx_seq_v)
        handles = []
        for j in range(SEQ_IDX // 128):
            handles.append(pltpu.async_copy(
                seq16_hbm.at[idx_seq_v.at[j]],
                seq_dst.at[pl.ds(j * 128, 128)], sem_seq))
        for h in handles:
            h.wait()

        def pool_row(r, inner):
            acc0 = jnp.zeros((16,), jnp.float32)
            acc1 = jnp.zeros((16,), jnp.float32)
            base = r * LSEQ
            for l in range(LSEQ):
                row16 = plsc.bitcast(seq_dst[base + l], jnp.bfloat16)
                a, b = plsc.unpack(row16, format=plsc.PackFormat.INTERLEAVED)
                acc0 = acc0 + a
                acc1 = acc1 + b
            cnt = jnp.zeros((16,), jnp.int32)
            row = r // 2
            col = (r % 2) * LSEQ
            for k in range(LSEQ // 16):
                ids = idx_seq_v[row, pl.ds(col + k * 16, 16)]
                cnt = cnt + jnp.minimum(jnp.abs(ids), 1)
            denom = jnp.sum(cnt).astype(jnp.float32) + jnp.float32(1e-16)
            pbuf[r] = plsc.bitcast(
                plsc.pack(acc0 / denom, acc1 / denom,
                          format=plsc.PackFormat.INTERLEAVED), jnp.int32)
            return inner

        lax.fori_loop(0, CHUNK, pool_row, 0)
        pltpu.sync_copy(pbuf, pooled_hbm.at[pl.ds(g * CHUNK, CHUNK)])
        return carry

    lax.fori_loop(0, chunks_per_w, do_chunk, 0)


def _kernel_b(idx_sp_hbm, tflat_hbm, pooled_hbm, out_hbm,
              sp_raw, sp_dst, idx_sp_v, pbuf, sem):
    info = plsc.get_sparse_core_info()
    nc = info.num_cores
    wid = lax.axis_index("s") * nc + lax.axis_index("c")
    n_workers = nc * info.num_subcores
    chunks_per_w = N_CHUNKS // n_workers

    def do_chunk(ci, carry):
        g = wid * chunks_per_w + ci
        pltpu.sync_copy(idx_sp_hbm.at[pl.ds(g * (SP_IDX_PAD // 128),
                                            SP_IDX_PAD // 128)], idx_sp_v)
        handles = []
        for j in range(SP_IDX_PAD // 128):
            handles.append(pltpu.async_copy(
                tflat_hbm.at[idx_sp_v.at[j]],
                sp_raw.at[pl.ds(j * 128, 128)], sem))
        pltpu.sync_copy(pooled_hbm.at[pl.ds(g * CHUNK, CHUNK)], pbuf)
        for h in handles:
            h.wait()

        def conv_row(p, inner):
            row16 = plsc.bitcast(sp_raw[p], jnp.bfloat16)
            a, b = plsc.unpack(row16, format=plsc.PackFormat.INTERLEAVED)
            sp_dst[p, pl.ds(0, 16)] = a
            sp_dst[p, pl.ds(16, 16)] = b
            return inner

        lax.fori_loop(0, SP_IDX, conv_row, 0)

        iota = lax.iota(jnp.int32, 16)

        def put_pooled(r, inner):
            row16 = plsc.bitcast(pbuf[r], jnp.bfloat16)
            a, b = plsc.unpack(row16, format=plsc.PackFormat.INTERLEAVED)
            slot = jnp.full((16,), r * SP_SLOTS + (SP_SLOTS - 1), jnp.int32)
            plsc.store_scatter(sp_dst, [slot, iota * 2], a)
            plsc.store_scatter(sp_dst, [slot, iota * 2 + 1], b)
            return inner

        lax.fori_loop(0, CHUNK, put_pooled, 0)
        pltpu.sync_copy(sp_dst.at[pl.ds(0, SP_IDX)],
                        out_hbm.at[pl.ds(g * SP_IDX, SP_IDX)])
        return carry

    lax.fori_loop(0, chunks_per_w, do_chunk, 0)


@jax.jit
def kernel(sparse_ids, seq_ids, sparse_tables, seq_table):
    ids32 = sparse_ids.astype(jnp.int32)
    seq32 = seq_ids.astype(jnp.int32)

    t2 = jnp.swapaxes(sparse_tables, 1, 2).reshape(F * D, V)
    cpad = sparse_tables[:, TAIL1:, :].reshape(F * 32, D)
    cpad = jnp.pad(cpad, ((0, 0), (0, 128 - D)))
    seq16 = lax.bitcast_convert_type(
        seq_table.astype(jnp.bfloat16).reshape(V, W, 2), jnp.int32)

    idx_sp = ids32 + (jnp.arange(F, dtype=jnp.int32) * VP)[None, :]
    idx_sp = jnp.pad(idx_sp, ((0, 0), (0, 1)))
    idx_sp = idx_sp.reshape(B // CHUNK, SP_IDX)
    idx_sp = jnp.pad(idx_sp, ((0, 0), (0, SP_IDX_PAD - SP_IDX)))
    idx_sp = idx_sp.reshape(-1, 128)

    idx_seq = jnp.pad(seq32, ((0, 0), (0, LSEQ - L)))
    idx_seq = idx_seq.reshape(-1, 128)

    run_a1 = pl.kernel(
        _kernel_a1,
        out_type=jax.ShapeDtypeStruct((F * VP // 8, 128), jnp.int32),
        mesh=plsc.VectorSubcoreMesh(core_axis_name="c", subcore_axis_name="s"),
        scratch_types=[
            pltpu.VMEM((D, VSPAN), jnp.float32),
            pltpu.VMEM((D, VSPAN), jnp.float32),
            pltpu.VMEM((VSPAN // 8, 128), jnp.int32),
            pltpu.VMEM((VSPAN // 8, 128), jnp.int32),
            pltpu.VMEM((D, 128), jnp.float32),
            pltpu.VMEM((16, 128), jnp.int32),
            pltpu.SemaphoreType.DMA,
            pltpu.SemaphoreType.DMA,
        ],
        compiler_params=pltpu.CompilerParams(needs_layout_passes=False),
    )
    tflat8 = run_a1(t2, cpad)

    run_a2 = pl.kernel(
        _kernel_a2,
        out_type=jax.ShapeDtypeStruct((B, W), jnp.int32),
        mesh=plsc.VectorSubcoreMesh(core_axis_name="c", subcore_axis_name="s"),
        scratch_types=[
            pltpu.VMEM((SEQ_IDX, W), jnp.int32),
            pltpu.VMEM((SEQ_IDX // 128, 128), jnp.int32),
            pltpu.VMEM((CHUNK, W), jnp.int32),
            pltpu.SemaphoreType.DMA,
        ],
        compiler_params=pltpu.CompilerParams(use_tc_tiling_on_sc=False,
                                             needs_layout_passes=False),
    )
    pooled = run_a2(idx_seq, seq16)

    tflat = tflat8.reshape(F * VP, W)

    run_b = pl.kernel(
        _kernel_b,
        out_type=jax.ShapeDtypeStruct((B * SP_SLOTS, D), jnp.float32),
        mesh=plsc.VectorSubcoreMesh(core_axis_name="c", subcore_axis_name="s"),
        scratch_types=[
            pltpu.VMEM((SP_IDX_PAD, W), jnp.int32),
            pltpu.VMEM((SP_IDX, D), jnp.float32),
            pltpu.VMEM((SP_IDX_PAD // 128, 128), jnp.int32),
            pltpu.VMEM((CHUNK, W), jnp.int32),
            pltpu.SemaphoreType.DMA,
        ],
        compiler_params=pltpu.CompilerParams(use_tc_tiling_on_sc=False,
                                             needs_layout_passes=False),
    )
    out = run_b(idx_sp, tflat, pooled)
    return out.reshape(B, F * D + D)


# speedup vs baseline: 1.0009x; 1.0009x over previous
import jax
import jax.numpy as jnp
from jax import lax
from jax.experimental import pallas as pl
from jax.experimental.pallas import tpu as pltpu, tpu_sc as plsc

B = 4096
F = 26
V = 100000
D = 32
L = 50
W = D // 2

LSEQ = 64
CHUNK = 32
SP_SLOTS = 27
SP_IDX = CHUNK * SP_SLOTS
SP_IDX_PAD = 1024
SEQ_IDX = CHUNK * LSEQ
N_CHUNKS = B // CHUNK

VSPAN = 1024
NFULL = V // VSPAN
TAIL0 = NFULL * VSPAN
TAIL1 = TAIL0 + 5 * 128
N_ITEMS = F * NFULL
VP = 100096
N_WORKERS = 32
ITEMS_PER_W = -(-N_ITEMS // N_WORKERS)


def _transpose_block(tb, ob, width, iota2a, iota2b):
    def tr(u, col):
        for j in range(8):
            a = plsc.load_gather(tb, [iota2a, col])
            bvals = plsc.load_gather(tb, [iota2b, col])
            ob[u, pl.ds(j * W, W)] = plsc.bitcast(
                plsc.pack(a, bvals, format=plsc.PackFormat.INTERLEAVED),
                jnp.int32)
            col = col + 1
        return col
    lax.fori_loop(0, width // 8, tr, jnp.zeros((16,), jnp.int32))


def _kernel_a1(t2_hbm, cpad_hbm, tflat_hbm,
               tb0, tb1, ob0, ob1, tbe, obe, sem_t, sem_o):
    info = plsc.get_sparse_core_info()
    nc = info.num_cores
    wid = lax.axis_index("s") * nc + lax.axis_index("c")
    iota = lax.iota(jnp.int32, 16)
    iota2a = iota
    iota2b = iota + 16

    @pl.when(wid < F)
    def _edge():
        f = wid
        for kk in range(5):
            pltpu.sync_copy(
                t2_hbm.at[pl.ds(f * D, D), pl.ds(TAIL0 + kk * 128, 128)],
                tbe.at[:, pl.ds(0, 128)])
            _transpose_block(tbe, obe, 128, iota2a, iota2b)
            pltpu.sync_copy(
                obe.at[pl.ds(0, 16)],
                tflat_hbm.at[pl.ds(pl.multiple_of(
                    f * (VP // 8) + (TAIL0 + kk * 128) // 8, 8), 16)])
        pltpu.sync_copy(cpad_hbm.at[pl.ds(f * 32, 32), pl.ds(0, 128)],
                        tbe.at[:, pl.ds(0, 128)])

        def crow(j, carry):
            jf = jnp.full((16,), j, jnp.int32)
            a = plsc.load_gather(tbe, [jf, iota2a])
            bvals = plsc.load_gather(tbe, [jf, iota2b])
            packed = plsc.bitcast(
                plsc.pack(a, bvals, format=plsc.PackFormat.INTERLEAVED),
                jnp.int32)
            obe[j // 8, pl.ds((j % 8) * W, W)] = packed
            return carry
        lax.fori_loop(0, 32, crow, 0)
        pltpu.sync_copy(obe.at[pl.ds(0, 4)],
                        tflat_hbm.at[pl.ds(pl.multiple_of(f * (VP // 8) + TAIL1 // 8, 8), 4)])

    def item_of(i):
        item = i * N_WORKERS + wid
        return jnp.minimum(item, N_ITEMS - 1), item < N_ITEMS

    def stage_item(i, tb):
        item, _ = item_of(i)
        f = item // NFULL
        s = item % NFULL
        for dt in range(4):
            pltpu.async_copy(
                t2_hbm.at[pl.ds(pl.multiple_of(f * D + dt * 8, 8), 8),
                          pl.ds(s * VSPAN, VSPAN)],
                tb.at[pl.ds(dt * 8, 8), pl.ds(0, VSPAN)], sem_t)

    def drain_tb(tb):
        for dt in range(4):
            pltpu.make_async_copy(
                t2_hbm.at[pl.ds(0, 8), pl.ds(0, VSPAN)],
                tb.at[pl.ds(dt * 8, 8), pl.ds(0, VSPAN)], sem_t).wait()

    def drain_ob(ob):
        pltpu.make_async_copy(
            tflat_hbm.at[pl.ds(0, VSPAN // 8)], ob, sem_o).wait()

    def do_item(i, tb, ob):
        item, valid = item_of(i)

        @pl.when(valid)
        def _():
            f = item // NFULL
            s = item % NFULL
            _transpose_block(tb, ob, VSPAN, iota2a, iota2b)
            pltpu.async_copy(
                ob, tflat_hbm.at[pl.ds(pl.multiple_of(f * (VP // 8) + s * (VSPAN // 8), 8),
                                   VSPAN // 8)],
                sem_o)

    stage_item(0, tb0)

    def pipe(k, carry):
        @pl.when(k % 2 == 0)
        def _ev():
            drain_tb(tb0)
            stage_item(k + 1, tb1)
            @pl.when(jnp.logical_and(k >= 2, (k - 2) * N_WORKERS + wid < N_ITEMS))
            def _d0():
                drain_ob(ob0)
            do_item(k, tb0, ob0)

        @pl.when(k % 2 == 1)
        def _od():
            drain_tb(tb1)
            stage_item(k + 1, tb0)
            @pl.when(jnp.logical_and(k >= 2, (k - 2) * N_WORKERS + wid < N_ITEMS))
            def _d1():
                drain_ob(ob1)
            do_item(k, tb1, ob1)

        return carry

    lax.fori_loop(0, ITEMS_PER_W, pipe, 0)
    if ITEMS_PER_W % 2 == 0:
        drain_tb(tb0)
    else:
        drain_tb(tb1)
    for kk in (ITEMS_PER_W - 2, ITEMS_PER_W - 1):
        @pl.when(kk * N_WORKERS + wid < N_ITEMS)
        def _dt(kk=kk):
            drain_ob(ob0 if kk % 2 == 0 else ob1)


def _kernel_a2(idx_seq_hbm, seq16_hbm, pooled_hbm,
               seq_dst, idx_seq_v, pbuf, sem_seq):
    info = plsc.get_sparse_core_info()
    nc = info.num_cores
    wid = lax.axis_index("s") * nc + lax.axis_index("c")
    n_workers = nc * info.num_subcores
    chunks_per_w = N_CHUNKS // n_workers

    def do_chunk(ci, carry):
        g = wid * chunks_per_w + ci
        pltpu.sync_copy(idx_seq_hbm.at[pl.ds(g * (SEQ_IDX // 128),
                                             SEQ_IDX // 128)], idx_seq_v)
        handles = []
        for j in range(SEQ_IDX // 128):
            handles.append(pltpu.async_copy(
                seq16_hbm.at[idx_seq_v.at[j]],
                seq_dst.at[pl.ds(j * 128, 128)], sem_seq))
        for h in handles:
            h.wait()

        def pool_row(r, inner):
            acc0 = jnp.zeros((16,), jnp.float32)
            acc1 = jnp.zeros((16,), jnp.float32)
            base = r * LSEQ
            for l in range(LSEQ):
                row16 = plsc.bitcast(seq_dst[base + l], jnp.bfloat16)
                a, b = plsc.unpack(row16, format=plsc.PackFormat.INTERLEAVED)
                acc0 = acc0 + a
                acc1 = acc1 + b
            cnt = jnp.zeros((16,), jnp.int32)
            row = r // 2
            col = (r % 2) * LSEQ
            for k in range(LSEQ // 16):
                ids = idx_seq_v[row, pl.ds(col + k * 16, 16)]
                cnt = cnt + jnp.minimum(jnp.abs(ids), 1)
            denom = jnp.sum(cnt).astype(jnp.float32) + jnp.float32(1e-16)
            pbuf[r] = plsc.bitcast(
                plsc.pack(acc0 / denom, acc1 / denom,
                          format=plsc.PackFormat.INTERLEAVED), jnp.int32)
            return inner

        lax.fori_loop(0, CHUNK, pool_row, 0)
        pltpu.sync_copy(pbuf, pooled_hbm.at[pl.ds(g * CHUNK, CHUNK)])
        return carry

    lax.fori_loop(0, chunks_per_w, do_chunk, 0)


def _kernel_b(idx_sp_hbm, tflat_hbm, pooled_hbm, out_hbm,
              sp_raw, sp_dst, idx_sp_v, pbuf, sem):
    info = plsc.get_sparse_core_info()
    nc = info.num_cores
    wid = lax.axis_index("s") * nc + lax.axis_index("c")
    n_workers = nc * info.num_subcores
    chunks_per_w = N_CHUNKS // n_workers

    def do_chunk(ci, carry):
        g = wid * chunks_per_w + ci
        pltpu.sync_copy(idx_sp_hbm.at[pl.ds(g * (SP_IDX_PAD // 128),
                                            SP_IDX_PAD // 128)], idx_sp_v)
        handles = []
        for j in range(SP_IDX_PAD // 128):
            handles.append(pltpu.async_copy(
                tflat_hbm.at[idx_sp_v.at[j]],
                sp_raw.at[pl.ds(j * 128, 128)], sem))
        pltpu.sync_copy(pooled_hbm.at[pl.ds(g * CHUNK, CHUNK)], pbuf)
        for h in handles:
            h.wait()

        def conv_row(p, inner):
            row16 = plsc.bitcast(sp_raw[p], jnp.bfloat16)
            a, b = plsc.unpack(row16, format=plsc.PackFormat.INTERLEAVED)
            sp_dst[p, pl.ds(0, 16)] = a
            sp_dst[p, pl.ds(16, 16)] = b
            return inner

        lax.fori_loop(0, SP_IDX, conv_row, 0)

        iota = lax.iota(jnp.int32, 16)

        def put_pooled(r, inner):
            row16 = plsc.bitcast(pbuf[r], jnp.bfloat16)
            a, b = plsc.unpack(row16, format=plsc.PackFormat.INTERLEAVED)
            slot = jnp.full((16,), r * SP_SLOTS + (SP_SLOTS - 1), jnp.int32)
            plsc.store_scatter(sp_dst, [slot, iota * 2], a)
            plsc.store_scatter(sp_dst, [slot, iota * 2 + 1], b)
            return inner

        lax.fori_loop(0, CHUNK, put_pooled, 0)
        pltpu.sync_copy(sp_dst.at[pl.ds(0, SP_IDX)],
                        out_hbm.at[pl.ds(g * SP_IDX, SP_IDX)])
        return carry

    lax.fori_loop(0, chunks_per_w, do_chunk, 0)


@jax.jit
def kernel(sparse_ids, seq_ids, sparse_tables, seq_table):
    ids32 = sparse_ids.astype(jnp.int32)
    seq32 = seq_ids.astype(jnp.int32)

    t2 = jnp.swapaxes(sparse_tables, 1, 2).reshape(F * D, V)
    cpad = sparse_tables[:, TAIL1:, :].reshape(F * 32, D)
    cpad = jnp.pad(cpad, ((0, 0), (0, 128 - D)))
    seq16 = lax.bitcast_convert_type(
        seq_table.astype(jnp.bfloat16).reshape(V, W, 2), jnp.int32)

    idx_sp = ids32 + (jnp.arange(F, dtype=jnp.int32) * VP)[None, :]
    idx_sp = jnp.pad(idx_sp, ((0, 0), (0, 1)))
    idx_sp = idx_sp.reshape(B // CHUNK, SP_IDX)
    idx_sp = jnp.pad(idx_sp, ((0, 0), (0, SP_IDX_PAD - SP_IDX)))
    idx_sp = idx_sp.reshape(-1, 128)

    idx_seq = jnp.pad(seq32, ((0, 0), (0, LSEQ - L)))
    idx_seq = idx_seq.reshape(-1, 128)

    run_a1 = pl.kernel(
        _kernel_a1,
        out_type=jax.ShapeDtypeStruct((F * VP // 8, 128), jnp.int32),
        mesh=plsc.VectorSubcoreMesh(core_axis_name="c", subcore_axis_name="s"),
        scratch_types=[
            pltpu.VMEM((D, VSPAN + 1), jnp.float32),
            pltpu.VMEM((D, VSPAN + 1), jnp.float32),
            pltpu.VMEM((VSPAN // 8, 128), jnp.int32),
            pltpu.VMEM((VSPAN // 8, 128), jnp.int32),
            pltpu.VMEM((D, 129), jnp.float32),
            pltpu.VMEM((16, 128), jnp.int32),
            pltpu.SemaphoreType.DMA,
            pltpu.SemaphoreType.DMA,
        ],
        compiler_params=pltpu.CompilerParams(needs_layout_passes=False),
    )
    tflat8 = run_a1(t2, cpad)

    run_a2 = pl.kernel(
        _kernel_a2,
        out_type=jax.ShapeDtypeStruct((B, W), jnp.int32),
        mesh=plsc.VectorSubcoreMesh(core_axis_name="c", subcore_axis_name="s"),
        scratch_types=[
            pltpu.VMEM((SEQ_IDX, W), jnp.int32),
            pltpu.VMEM((SEQ_IDX // 128, 128), jnp.int32),
            pltpu.VMEM((CHUNK, W), jnp.int32),
            pltpu.SemaphoreType.DMA,
        ],
        compiler_params=pltpu.CompilerParams(use_tc_tiling_on_sc=False,
                                             needs_layout_passes=False),
    )
    pooled = run_a2(idx_seq, seq16)

    tflat = tflat8.reshape(F * VP, W)

    run_b = pl.kernel(
        _kernel_b,
        out_type=jax.ShapeDtypeStruct((B * SP_SLOTS, D), jnp.float32),
        mesh=plsc.VectorSubcoreMesh(core_axis_name="c", subcore_axis_name="s"),
        scratch_types=[
            pltpu.VMEM((SP_IDX_PAD, W), jnp.int32),
            pltpu.VMEM((SP_IDX, D), jnp.float32),
            pltpu.VMEM((SP_IDX_PAD // 128, 128), jnp.int32),
            pltpu.VMEM((CHUNK, W), jnp.int32),
            pltpu.SemaphoreType.DMA,
        ],
        compiler_params=pltpu.CompilerParams(use_tc_tiling_on_sc=False,
                                             needs_layout_passes=False),
    )
    out = run_b(idx_sp, tflat, pooled)
    return out.reshape(B, F * D + D)


# speedup vs baseline: 1.2989x; 1.2977x over previous
import jax
import jax.numpy as jnp
from jax import lax
from jax.experimental import pallas as pl
from jax.experimental.pallas import tpu as pltpu, tpu_sc as plsc

B = 4096
F = 26
V = 100000
D = 32
L = 50

LSEQ = 64
CHUNK = 32
SP_SLOTS = 27
SP_IDX = CHUNK * SP_SLOTS
SP_IDX_PAD = 1024
SEQ_IDX = CHUNK * LSEQ
N_CHUNKS = B // CHUNK


def _sc_kernel_body(idx_sp_hbm, idx_seq_hbm, tables_hbm, seq_table_hbm,
                    out_hbm, sp_dst, seq_dst, idx_sp_v, idx_seq_v, sem):
    info = plsc.get_sparse_core_info()
    nc = info.num_cores
    wid = lax.axis_index("s") * nc + lax.axis_index("c")
    n_workers = nc * info.num_subcores
    chunks_per_w = N_CHUNKS // n_workers

    def do_chunk(ci, carry):
        g = wid * chunks_per_w + ci
        with jax.named_scope("stage_idx"):
            pltpu.sync_copy(idx_sp_hbm.at[pl.ds(g * (SP_IDX_PAD // 128),
                                                SP_IDX_PAD // 128)], idx_sp_v)
            pltpu.sync_copy(idx_seq_hbm.at[pl.ds(g * (SEQ_IDX // 128),
                                                 SEQ_IDX // 128)], idx_seq_v)
        with jax.named_scope("gathers"):
            handles = []
            for j in range(SP_IDX_PAD // 128):
                handles.append(pltpu.async_copy(
                    tables_hbm.at[idx_sp_v.at[j]],
                    sp_dst.at[pl.ds(j * 128, 128)], sem))
            for j in range(SEQ_IDX // 128):
                handles.append(pltpu.async_copy(
                    seq_table_hbm.at[idx_seq_v.at[j]],
                    seq_dst.at[pl.ds(j * 128, 128)], sem))
            for h in handles:
                h.wait()

        def pool_row(r, inner):
            acc0 = jnp.zeros((16,), jnp.float32)
            acc1 = jnp.zeros((16,), jnp.float32)
            base = r * LSEQ
            for l in range(LSEQ):
                acc0 = acc0 + seq_dst[base + l, pl.ds(0, 16)]
                acc1 = acc1 + seq_dst[base + l, pl.ds(16, 16)]
            cnt = jnp.zeros((16,), jnp.int32)
            row = r // 2
            col = (r % 2) * LSEQ
            for k in range(LSEQ // 16):
                ids = idx_seq_v[row, pl.ds(col + k * 16, 16)]
                cnt = cnt + jnp.minimum(jnp.abs(ids), 1)
            denom = jnp.sum(cnt).astype(jnp.float32) + jnp.float32(1e-16)
            slot = r * SP_SLOTS + (SP_SLOTS - 1)
            sp_dst[slot, pl.ds(0, 16)] = acc0 / denom
            sp_dst[slot, pl.ds(16, 16)] = acc1 / denom
            return inner

        with jax.named_scope("pool"):
            lax.fori_loop(0, CHUNK, pool_row, 0)

        with jax.named_scope("store"):
            pltpu.sync_copy(sp_dst.at[pl.ds(0, SP_IDX)],
                            out_hbm.at[pl.ds(g * SP_IDX, SP_IDX)])
        return carry

    lax.fori_loop(0, chunks_per_w, do_chunk, 0)


@jax.jit
def kernel(sparse_ids, seq_ids, sparse_tables, seq_table):
    ids32 = sparse_ids.astype(jnp.int32)
    seq32 = seq_ids.astype(jnp.int32)

    idx_sp = ids32 + (jnp.arange(F, dtype=jnp.int32) * V)[None, :]
    idx_sp = jnp.pad(idx_sp, ((0, 0), (0, 1)))
    idx_sp = idx_sp.reshape(B // CHUNK, SP_IDX)
    idx_sp = jnp.pad(idx_sp, ((0, 0), (0, SP_IDX_PAD - SP_IDX)))
    idx_sp = idx_sp.reshape(-1, 128)

    idx_seq = jnp.pad(seq32, ((0, 0), (0, LSEQ - L)))
    idx_seq = idx_seq.reshape(-1, 128)

    tables_flat = sparse_tables.reshape(F * V, D)

    run = pl.kernel(
        _sc_kernel_body,
        out_type=jax.ShapeDtypeStruct((B * SP_SLOTS, D), jnp.float32),
        mesh=plsc.VectorSubcoreMesh(core_axis_name="c", subcore_axis_name="s"),
        compiler_params=pltpu.CompilerParams(use_tc_tiling_on_sc=False,
                                             needs_layout_passes=False),
        scratch_types=[
            pltpu.VMEM((SP_IDX_PAD, D), jnp.float32),
            pltpu.VMEM((SEQ_IDX, D), jnp.float32),
            pltpu.VMEM((SP_IDX_PAD // 128, 128), jnp.int32),
            pltpu.VMEM((SEQ_IDX // 128, 128), jnp.int32),
            pltpu.SemaphoreType.DMA,
        ],
    )
    out = run(idx_sp, idx_seq, tables_flat, seq_table)
    return out.reshape(B, F * D + D)
